# baseline (speedup 1.0000x reference)
import functools

import jax
import jax.numpy as jnp
from jax import lax
from jax.experimental import pallas as pl
from jax.experimental.pallas import tpu as pltpu
from jax.experimental.pallas import tpu_sc as plsc

NUM_STAGES = 8
K = 1024
D = 256
N_ROWS = 4096
ROWS_PER_BLOCK = 512


def _scores_to_idx(r, cb, first, idx_ref, dsum_ref):
    s = lax.dot_general(r, cb, (((1,), (1,)), ((), ())),
                        preferred_element_type=jnp.float32)
    a = jnp.sum(r * r, axis=1, keepdims=True)
    cn = jnp.sum(cb * cb, axis=1)[None, :]
    d = (a - 2.0 * s) + cn
    dmin = jnp.min(d, axis=1, keepdims=True)
    ids = lax.broadcasted_iota(jnp.int32, d.shape, 1)
    idx_ref[...] = jnp.min(jnp.where(d == dmin, ids, K), axis=1, keepdims=True)
    row = jnp.full((1, 128), jnp.sum(dmin), jnp.float32)
    prev = jnp.where(first, jnp.zeros_like(row), dsum_ref[...])
    dsum_ref[...] = prev + row


def _score_kernel(r_ref, cb_ref, idx_ref, dsum_ref):
    first = pl.program_id(0) == 0
    _scores_to_idx(r_ref[...], cb_ref[...], first, idx_ref, dsum_ref)


def _update_score_kernel(r_ref, e_ref, cb_ref, *refs, tap):
    first = pl.program_id(0) == 0
    r = r_ref[...]
    e = e_ref[...]
    q = r + (e - r)
    rn = r - q
    if tap:
        rout_ref, qtap_ref, idx_ref, dsum_ref = refs
        qtap_ref[...] = q
    else:
        rout_ref, idx_ref, dsum_ref = refs
    rout_ref[...] = rn
    _scores_to_idx(rn, cb_ref[...], first, idx_ref, dsum_ref)


def _final_kernel(h_ref, r_ref, e_ref, out_ref):
    r = r_ref[...]
    q = r + (e_ref[...] - r)
    rn = r - q
    out_ref[...] = h_ref[...] - rn


def _make_sc_gather():
    info = plsc.get_sparse_core_info()
    nc, ns = info.num_cores, info.num_subcores
    bpw = N_ROWS // (nc * ns)
    mesh = plsc.VectorSubcoreMesh(core_axis_name="c", subcore_axis_name="s")

    @functools.partial(
        pl.kernel,
        out_type=jax.ShapeDtypeStruct((N_ROWS, D), jnp.float32),
        scratch_types=[
            pltpu.VMEM((bpw,), jnp.int32),
            pltpu.VMEM((bpw, D), jnp.float32),
            pltpu.SemaphoreType.DMA,
        ],
        mesh=mesh,
    )
    def sc_gather(table_hbm, idx_hbm, out_hbm, idx_v, rows_v, sem):
        wid = lax.axis_index("s") * nc + lax.axis_index("c")
        base = wid * bpw
        pltpu.sync_copy(idx_hbm.at[pl.ds(base, bpw)], idx_v)
        pltpu.async_copy(table_hbm.at[idx_v], rows_v, sem).wait()
        pltpu.sync_copy(rows_v, out_hbm.at[pl.ds(base, bpw)])

    return sc_gather


def kernel(x, codebooks):
    B, Dx, T = x.shape
    h = jnp.transpose(x, (0, 2, 1)).reshape(N_ROWS, D)

    grid = (N_ROWS // ROWS_PER_BLOCK,)
    row_spec = pl.BlockSpec((ROWS_PER_BLOCK, D), lambda c: (c, 0))
    idx_spec = pl.BlockSpec((ROWS_PER_BLOCK, 1), lambda c: (c, 0))
    cb_spec = pl.BlockSpec((K, D), lambda c: (0, 0))
    dsum_spec = pl.BlockSpec((1, 128), lambda c: (0, 0))
    row_sh = jax.ShapeDtypeStruct((N_ROWS, D), jnp.float32)
    idx_sh = jax.ShapeDtypeStruct((N_ROWS, 1), jnp.int32)
    dsum_sh = jax.ShapeDtypeStruct((1, 128), jnp.float32)

    score0 = pl.pallas_call(
        _score_kernel, grid=grid,
        in_specs=[row_spec, cb_spec],
        out_specs=[idx_spec, dsum_spec],
        out_shape=[idx_sh, dsum_sh])

    def make_step(tap):
        outs = ([row_spec, row_spec, idx_spec, dsum_spec] if tap
                else [row_spec, idx_spec, dsum_spec])
        shs = ([row_sh, row_sh, idx_sh, dsum_sh] if tap
               else [row_sh, idx_sh, dsum_sh])
        return pl.pallas_call(
            functools.partial(_update_score_kernel, tap=tap), grid=grid,
            in_specs=[row_spec, row_spec, cb_spec],
            out_specs=outs, out_shape=shs)

    final = pl.pallas_call(
        _final_kernel, grid=grid,
        in_specs=[row_spec, row_spec, row_spec],
        out_specs=row_spec, out_shape=row_sh)

    sc_gather = _make_sc_gather()

    idx, dsum0 = score0(h, codebooks[0])
    e = sc_gather(codebooks[0], idx.reshape(N_ROWS))
    r = h
    taps = []
    dsums = [dsum0]
    for i in range(1, NUM_STAGES):
        tap = (i - 1) in (0, 1)
        outs = make_step(tap)(r, e, codebooks[i])
        if tap:
            r, q, idx, ds = outs
            taps.append(q)
        else:
            r, idx, ds = outs
        dsums.append(ds)
        e = sc_gather(codebooks[i], idx.reshape(N_ROWS))
    out = final(h, r, e)

    def back(y):
        return jnp.transpose(y.reshape(B, T, Dx), (0, 2, 1))

    commits = jnp.stack([ds[0, 0] for ds in dsums]) / jnp.float32(N_ROWS * D)
    com = jnp.mean(commits)
    return (back(out), back(taps[0]), back(taps[1]), com)


# speedup vs baseline: 1.0148x; 1.0148x over previous
import functools

import jax
import jax.numpy as jnp
from jax import lax
from jax.experimental import pallas as pl
from jax.experimental.pallas import tpu as pltpu
from jax.experimental.pallas import tpu_sc as plsc

NUM_STAGES = 8
K = 1024
D = 256
N_ROWS = 4096
ROWS_PER_BLOCK = 512


def _scores_to_idx(r, cb, first, idx_ref, dsum_ref):
    s = lax.dot_general(r, cb, (((1,), (1,)), ((), ())),
                        preferred_element_type=jnp.float32)
    a = jnp.sum(r * r, axis=1, keepdims=True)
    cn = jnp.sum(cb * cb, axis=1)[None, :]
    d = (a - 2.0 * s) + cn
    dmin = jnp.min(d, axis=1, keepdims=True)
    ids = lax.broadcasted_iota(jnp.int32, d.shape, 1)
    idx_ref[...] = jnp.min(jnp.where(d == dmin, ids, K), axis=1, keepdims=True)
    row = jnp.full((1, 128), jnp.sum(dmin), jnp.float32)
    prev = jnp.where(first, jnp.zeros_like(row), dsum_ref[...])
    dsum_ref[...] = prev + row


def _score_kernel(r_ref, cb_ref, idx_ref, dsum_ref):
    first = pl.program_id(0) == 0
    _scores_to_idx(r_ref[...], cb_ref[...], first, idx_ref, dsum_ref)


def _update_score_kernel(r_ref, e_ref, cb_ref, *refs, tap):
    first = pl.program_id(0) == 0
    r = r_ref[...]
    e = e_ref[...]
    q = r + (e - r)
    rn = r - q
    if tap:
        rout_ref, qtap_ref, idx_ref, dsum_ref = refs
        qtap_ref[...] = q
    else:
        rout_ref, idx_ref, dsum_ref = refs
    rout_ref[...] = rn
    _scores_to_idx(rn, cb_ref[...], first, idx_ref, dsum_ref)


def _final_kernel(h_ref, r_ref, e_ref, out_ref):
    r = r_ref[...]
    q = r + (e_ref[...] - r)
    rn = r - q
    out_ref[...] = h_ref[...] - rn


N_CHUNKS = 2
CHUNK = N_ROWS // N_CHUNKS


def _make_sc_gather(n_rows):
    info = plsc.get_sparse_core_info()
    nc, ns = info.num_cores, info.num_subcores
    bpw = n_rows // (nc * ns)
    mesh = plsc.VectorSubcoreMesh(core_axis_name="c", subcore_axis_name="s")

    @functools.partial(
        pl.kernel,
        out_type=jax.ShapeDtypeStruct((n_rows, D), jnp.float32),
        scratch_types=[
            pltpu.VMEM((bpw,), jnp.int32),
            pltpu.VMEM((bpw, D), jnp.float32),
            pltpu.SemaphoreType.DMA,
        ],
        mesh=mesh,
    )
    def sc_gather(table_hbm, idx_hbm, out_hbm, idx_v, rows_v, sem):
        wid = lax.axis_index("s") * nc + lax.axis_index("c")
        base = wid * bpw
        pltpu.sync_copy(idx_hbm.at[pl.ds(base, bpw)], idx_v)
        pltpu.async_copy(table_hbm.at[idx_v], rows_v, sem).wait()
        pltpu.sync_copy(rows_v, out_hbm.at[pl.ds(base, bpw)])

    return sc_gather


def kernel(x, codebooks):
    B, Dx, T = x.shape
    h = jnp.transpose(x, (0, 2, 1)).reshape(N_ROWS, D)

    grid = (CHUNK // ROWS_PER_BLOCK,)
    row_spec = pl.BlockSpec((ROWS_PER_BLOCK, D), lambda c: (c, 0))
    idx_spec = pl.BlockSpec((ROWS_PER_BLOCK, 1), lambda c: (c, 0))
    cb_spec = pl.BlockSpec((K, D), lambda c: (0, 0))
    dsum_spec = pl.BlockSpec((1, 128), lambda c: (0, 0))
    row_sh = jax.ShapeDtypeStruct((CHUNK, D), jnp.float32)
    idx_sh = jax.ShapeDtypeStruct((CHUNK, 1), jnp.int32)
    dsum_sh = jax.ShapeDtypeStruct((1, 128), jnp.float32)

    score0 = pl.pallas_call(
        _score_kernel, grid=grid,
        in_specs=[row_spec, cb_spec],
        out_specs=[idx_spec, dsum_spec],
        out_shape=[idx_sh, dsum_sh])

    def make_step(tap):
        outs = ([row_spec, row_spec, idx_spec, dsum_spec] if tap
                else [row_spec, idx_spec, dsum_spec])
        shs = ([row_sh, row_sh, idx_sh, dsum_sh] if tap
               else [row_sh, idx_sh, dsum_sh])
        return pl.pallas_call(
            functools.partial(_update_score_kernel, tap=tap), grid=grid,
            in_specs=[row_spec, row_spec, cb_spec],
            out_specs=outs, out_shape=shs)

    final = pl.pallas_call(
        _final_kernel, grid=grid,
        in_specs=[row_spec, row_spec, row_spec],
        out_specs=row_spec, out_shape=row_sh)

    sc_gather = _make_sc_gather(CHUNK)

    hs = [h[c * CHUNK:(c + 1) * CHUNK] for c in range(N_CHUNKS)]
    r = list(hs)
    e = [None] * N_CHUNKS
    taps = [[] for _ in range(N_CHUNKS)]
    dsums = []
    for c in range(N_CHUNKS):
        idx, ds = score0(hs[c], codebooks[0])
        dsums.append(ds)
        e[c] = sc_gather(codebooks[0], idx.reshape(CHUNK))
    for i in range(1, NUM_STAGES):
        tap = (i - 1) in (0, 1)
        step = make_step(tap)
        for c in range(N_CHUNKS):
            outs = step(r[c], e[c], codebooks[i])
            if tap:
                r[c], q, idx, ds = outs
                taps[c].append(q)
            else:
                r[c], idx, ds = outs
            dsums.append(ds)
            e[c] = sc_gather(codebooks[i], idx.reshape(CHUNK))
    out = jnp.concatenate([final(hs[c], r[c], e[c])
                           for c in range(N_CHUNKS)], axis=0)
    q1 = jnp.concatenate([taps[c][0] for c in range(N_CHUNKS)], axis=0)
    q2 = jnp.concatenate([taps[c][1] for c in range(N_CHUNKS)], axis=0)

    def back(y):
        return jnp.transpose(y.reshape(B, T, Dx), (0, 2, 1))

    commits = sum(ds[0, 0] for ds in dsums) / jnp.float32(
        N_ROWS * D * NUM_STAGES)
    com = commits
    return (back(out), back(q1), back(q2), com)
